# Initial kernel scaffold; baseline (speedup 1.0000x reference)
import jax
import jax.numpy as jnp
from jax.experimental import pallas as pl


def kernel(x, conv0_w, conv0_s, conv0_b, layer1_0_w1, layer1_0_s1, layer1_0_b1, layer1_0_w2, layer1_0_s2, layer1_0_b2, layer1_1_w1, layer1_1_s1, layer1_1_b1, layer1_1_w2, layer1_1_s2, layer1_1_b2, layer2_0_w1, layer2_0_s1, layer2_0_b1, layer2_0_w2, layer2_0_s2, layer2_0_b2, layer2_0_wd, layer2_0_sd, layer2_0_bd, layer2_1_w1, layer2_1_s1, layer2_1_b1, layer2_1_w2, layer2_1_s2, layer2_1_b2, layer3_0_w1, layer3_0_s1, layer3_0_b1, layer3_0_w2, layer3_0_s2, layer3_0_b2, layer3_0_wd, layer3_0_sd, layer3_0_bd, layer3_1_w1, layer3_1_s1, layer3_1_b1, layer3_1_w2, layer3_1_s2, layer3_1_b2, layer4_0_w1, layer4_0_s1, layer4_0_b1, layer4_0_w2, layer4_0_s2, layer4_0_b2, layer4_0_wd, layer4_0_sd, layer4_0_bd, layer4_1_w1, layer4_1_s1, layer4_1_b1, layer4_1_w2, layer4_1_s2, layer4_1_b2, fc1_w, fc1_s, fc1_b, fc2_w, fc2_b):
    raise NotImplementedError("write your pallas kernel here")


# speedup vs baseline: 52.6494x; 52.6494x over previous
import jax
import jax.numpy as jnp
from jax.experimental import pallas as pl
from jax.experimental.pallas import tpu as pltpu

BF16 = jnp.bfloat16
F32 = jnp.float32
NUM_CLASSES = 751
VMEM_LIMIT = 56 * 1024 * 1024


def _cparams():
    return pltpu.CompilerParams(
        dimension_semantics=("parallel",),
        vmem_limit_bytes=VMEM_LIMIT,
    )


def _conv3x3_bn(x, w, s, b, stride=1, res=None, relu=True, pad_scratch=None):
    B, H, W, C = x.shape
    Ho = (H - 1) // stride + 1
    Wo = (W - 1) // stride + 1
    taps = []
    if stride == 1:
        xp = jnp.pad(x, ((0, 0), (1, 1), (1, 1), (0, 0)))
        for dy in range(3):
            for dx in range(3):
                taps.append(xp[:, dy:dy + H, dx:dx + W, :])
    else:
        nc = len(pad_scratch)
        csz = C // nc
        for k in range(nc):
            pad_scratch[k][...] = jnp.pad(
                x[..., k * csz:(k + 1) * csz].astype(F32),
                ((0, 0), (1, 1), (1, 1), (0, 0)))
        for dy in range(3):
            for dx in range(3):
                parts = [
                    pad_scratch[k][:, dy:dy + 2 * (Ho - 1) + 1:2,
                                   dx:dx + 2 * (Wo - 1) + 1:2, :].astype(BF16)
                    for k in range(nc)]
                taps.append(parts[0] if nc == 1
                            else jnp.concatenate(parts, axis=-1))
    p = jnp.concatenate(taps, axis=-1).reshape(B * Ho * Wo, 9 * C)
    acc = jnp.dot(p, w, preferred_element_type=F32)
    acc = acc * s + b
    if res is not None:
        acc = acc + res
    if relu:
        acc = jnp.maximum(acc, 0.0)
    return acc.astype(BF16).reshape(B, Ho, Wo, w.shape[1])


def _basic_block(x, w1, s1, b1, w2, s2, b2, wd=None, sd=None, bd=None,
                 stride=1, pad_scratch=None):
    B, H, W, C = x.shape
    h = _conv3x3_bn(x, w1, s1, b1, stride=stride, pad_scratch=pad_scratch)
    _, Ho, Wo, Co = h.shape
    if wd is None:
        res = x.reshape(B * H * W, C).astype(F32)
    else:
        parts = [ps[:, 1:2 * (Ho - 1) + 2:2, 1:2 * (Wo - 1) + 2:2,
                    :].astype(BF16) for ps in pad_scratch]
        xs = parts[0] if len(parts) == 1 else jnp.concatenate(parts, axis=-1)
        res = jnp.dot(xs.reshape(B * Ho * Wo, C), wd,
                      preferred_element_type=F32) * sd + bd
    return _conv3x3_bn(h, w2, s2, b2, stride=1, res=res)


def _k1_body(p_ref, w_ref, s_ref, b_ref, o_ref, yp_ref):
    B = o_ref.shape[0]
    p = p_ref[...].reshape(B * 8192, 27)
    acc = jnp.dot(p, w_ref[...], preferred_element_type=F32)
    acc = jnp.maximum(acc * s_ref[...] + b_ref[...], 0.0)
    y = acc.reshape(B, 128, 64, 64)
    yp_ref[...] = jnp.pad(y, ((0, 0), (1, 1), (1, 1), (0, 0)))
    out = None
    for dy in range(3):
        for dx in range(3):
            t = yp_ref[:, dy:dy + 127:2, dx:dx + 63:2, :]
            out = t if out is None else jnp.maximum(out, t)
    o_ref[...] = out.astype(BF16)


def _k2_body(x_ref,
             a_w1, a_s1, a_b1, a_w2, a_s2, a_b2,
             c_w1, c_s1, c_b1, c_w2, c_s2, c_b2,
             d_w1, d_s1, d_b1, d_w2, d_s2, d_b2, d_wd, d_sd, d_bd,
             e_w1, e_s1, e_b1, e_w2, e_s2, e_b2,
             o_ref, pad2_ref):
    x = x_ref[...]
    x = _basic_block(x, a_w1[...], a_s1[...], a_b1[...],
                     a_w2[...], a_s2[...], a_b2[...])
    x = _basic_block(x, c_w1[...], c_s1[...], c_b1[...],
                     c_w2[...], c_s2[...], c_b2[...])
    x = _basic_block(x, d_w1[...], d_s1[...], d_b1[...],
                     d_w2[...], d_s2[...], d_b2[...],
                     wd=d_wd[...], sd=d_sd[...], bd=d_bd[...],
                     stride=2, pad_scratch=[pad2_ref])
    x = _basic_block(x, e_w1[...], e_s1[...], e_b1[...],
                     e_w2[...], e_s2[...], e_b2[...])
    o_ref[...] = x


def _k3_body(x_ref,
             a_w1, a_s1, a_b1, a_w2, a_s2, a_b2, a_wd, a_sd, a_bd,
             c_w1, c_s1, c_b1, c_w2, c_s2, c_b2,
             d_w1, d_s1, d_b1, d_w2, d_s2, d_b2, d_wd, d_sd, d_bd,
             e_w1, e_s1, e_b1, e_w2, e_s2, e_b2,
             o_ref, pad3_ref, pad4a_ref, pad4b_ref):
    x = x_ref[...]
    x = _basic_block(x, a_w1[...], a_s1[...], a_b1[...],
                     a_w2[...], a_s2[...], a_b2[...],
                     wd=a_wd[...], sd=a_sd[...], bd=a_bd[...],
                     stride=2, pad_scratch=[pad3_ref])
    x = _basic_block(x, c_w1[...], c_s1[...], c_b1[...],
                     c_w2[...], c_s2[...], c_b2[...])
    x = _basic_block(x, d_w1[...], d_s1[...], d_b1[...],
                     d_w2[...], d_s2[...], d_b2[...],
                     wd=d_wd[...], sd=d_sd[...], bd=d_bd[...],
                     stride=2, pad_scratch=[pad4a_ref, pad4b_ref])
    x = _basic_block(x, e_w1[...], e_s1[...], e_b1[...],
                     e_w2[...], e_s2[...], e_b2[...])
    o_ref[...] = jnp.mean(x.astype(F32), axis=(1, 2))


def _k4_body(x_ref, w1_ref, s1_ref, b1_ref, w2_ref, b2_ref, o_ref):
    h = jnp.dot(x_ref[...].astype(BF16), w1_ref[...],
                preferred_element_type=F32)
    h = jnp.maximum(h * s1_ref[...] + b1_ref[...], 0.0)
    out = jnp.dot(h.astype(BF16), w2_ref[...], preferred_element_type=F32)
    o_ref[...] = out + b2_ref[...]


def _wspec(shape):
    n = len(shape)
    return pl.BlockSpec(shape, lambda i, _n=n: (0,) * _n)


def kernel(x, conv0_w, conv0_s, conv0_b,
           layer1_0_w1, layer1_0_s1, layer1_0_b1,
           layer1_0_w2, layer1_0_s2, layer1_0_b2,
           layer1_1_w1, layer1_1_s1, layer1_1_b1,
           layer1_1_w2, layer1_1_s2, layer1_1_b2,
           layer2_0_w1, layer2_0_s1, layer2_0_b1,
           layer2_0_w2, layer2_0_s2, layer2_0_b2,
           layer2_0_wd, layer2_0_sd, layer2_0_bd,
           layer2_1_w1, layer2_1_s1, layer2_1_b1,
           layer2_1_w2, layer2_1_s2, layer2_1_b2,
           layer3_0_w1, layer3_0_s1, layer3_0_b1,
           layer3_0_w2, layer3_0_s2, layer3_0_b2,
           layer3_0_wd, layer3_0_sd, layer3_0_bd,
           layer3_1_w1, layer3_1_s1, layer3_1_b1,
           layer3_1_w2, layer3_1_s2, layer3_1_b2,
           layer4_0_w1, layer4_0_s1, layer4_0_b1,
           layer4_0_w2, layer4_0_s2, layer4_0_b2,
           layer4_0_wd, layer4_0_sd, layer4_0_bd,
           layer4_1_w1, layer4_1_s1, layer4_1_b1,
           layer4_1_w2, layer4_1_s2, layer4_1_b2,
           fc1_w, fc1_s, fc1_b, fc2_w, fc2_b):
    N = x.shape[0]

    xt = jnp.transpose(x, (0, 2, 3, 1)).astype(BF16)
    xpad = jnp.pad(xt, ((0, 0), (1, 1), (1, 1), (0, 0)))
    cols = [xpad[:, dy:dy + 128, dx:dx + 64, :]
            for dy in range(3) for dx in range(3)]
    p0 = jnp.concatenate(cols, axis=-1).reshape(N, 8192, 27)

    b1g = 1
    y1 = pl.pallas_call(
        _k1_body,
        grid=(N // b1g,),
        in_specs=[
            pl.BlockSpec((b1g, 8192, 27), lambda i: (i, 0, 0)),
            _wspec((27, 64)), _wspec((1, 64)), _wspec((1, 64)),
        ],
        out_specs=pl.BlockSpec((b1g, 64, 32, 64), lambda i: (i, 0, 0, 0)),
        out_shape=jax.ShapeDtypeStruct((N, 64, 32, 64), BF16),
        scratch_shapes=[pltpu.VMEM((b1g, 130, 66, 64), F32)],
        compiler_params=_cparams(),
    )(p0, conv0_w, conv0_s, conv0_b)

    b2g = 2
    l2_args = [
        layer1_0_w1, layer1_0_s1, layer1_0_b1,
        layer1_0_w2, layer1_0_s2, layer1_0_b2,
        layer1_1_w1, layer1_1_s1, layer1_1_b1,
        layer1_1_w2, layer1_1_s2, layer1_1_b2,
        layer2_0_w1, layer2_0_s1, layer2_0_b1,
        layer2_0_w2, layer2_0_s2, layer2_0_b2,
        layer2_0_wd, layer2_0_sd, layer2_0_bd,
        layer2_1_w1, layer2_1_s1, layer2_1_b1,
        layer2_1_w2, layer2_1_s2, layer2_1_b2,
    ]
    y2 = pl.pallas_call(
        _k2_body,
        grid=(N // b2g,),
        in_specs=[pl.BlockSpec((b2g, 64, 32, 64), lambda i: (i, 0, 0, 0))]
        + [_wspec(a.shape) for a in l2_args],
        out_specs=pl.BlockSpec((b2g, 32, 16, 128), lambda i: (i, 0, 0, 0)),
        out_shape=jax.ShapeDtypeStruct((N, 32, 16, 128), BF16),
        scratch_shapes=[pltpu.VMEM((b2g, 66, 34, 64), F32)],
        compiler_params=_cparams(),
    )(y1, *l2_args)

    b3g = 8
    l3_args = [
        layer3_0_w1, layer3_0_s1, layer3_0_b1,
        layer3_0_w2, layer3_0_s2, layer3_0_b2,
        layer3_0_wd, layer3_0_sd, layer3_0_bd,
        layer3_1_w1, layer3_1_s1, layer3_1_b1,
        layer3_1_w2, layer3_1_s2, layer3_1_b2,
        layer4_0_w1, layer4_0_s1, layer4_0_b1,
        layer4_0_w2, layer4_0_s2, layer4_0_b2,
        layer4_0_wd, layer4_0_sd, layer4_0_bd,
        layer4_1_w1, layer4_1_s1, layer4_1_b1,
        layer4_1_w2, layer4_1_s2, layer4_1_b2,
    ]
    feat = pl.pallas_call(
        _k3_body,
        grid=(N // b3g,),
        in_specs=[pl.BlockSpec((b3g, 32, 16, 128), lambda i: (i, 0, 0, 0))]
        + [_wspec(a.shape) for a in l3_args],
        out_specs=pl.BlockSpec((b3g, 512), lambda i: (i, 0)),
        out_shape=jax.ShapeDtypeStruct((N, 512), F32),
        scratch_shapes=[pltpu.VMEM((b3g, 34, 18, 128), F32),
                        pltpu.VMEM((b3g, 18, 10, 128), F32),
                        pltpu.VMEM((b3g, 18, 10, 128), F32)],
        compiler_params=_cparams(),
    )(y2, *l3_args)

    tm = N // 2
    logits = pl.pallas_call(
        _k4_body,
        grid=(2,),
        in_specs=[
            pl.BlockSpec((tm, 512), lambda i: (i, 0)),
            _wspec(fc1_w.shape), _wspec(fc1_s.shape), _wspec(fc1_b.shape),
            _wspec(fc2_w.shape), _wspec(fc2_b.shape),
        ],
        out_specs=pl.BlockSpec((tm, fc2_w.shape[1]), lambda i: (i, 0)),
        out_shape=jax.ShapeDtypeStruct((N, fc2_w.shape[1]), F32),
        compiler_params=_cparams(),
    )(feat, fc1_w, fc1_s, fc1_b, fc2_w, fc2_b)

    return logits[:, :NUM_CLASSES]


# speedup vs baseline: 55.4507x; 1.0532x over previous
import jax
import jax.numpy as jnp
from jax.experimental import pallas as pl
from jax.experimental.pallas import tpu as pltpu

BF16 = jnp.bfloat16
F32 = jnp.float32
NUM_CLASSES = 751
VMEM_LIMIT = 56 * 1024 * 1024


def _cparams():
    return pltpu.CompilerParams(
        dimension_semantics=("parallel",),
        vmem_limit_bytes=VMEM_LIMIT,
    )


def _conv3x3_bn(x, w, s, b, stride=1, res=None, relu=True, pad_scratch=None):
    B, H, W, C = x.shape
    Ho = (H - 1) // stride + 1
    Wo = (W - 1) // stride + 1
    taps = []
    if stride == 1:
        xp = jnp.pad(x, ((0, 0), (1, 1), (1, 1), (0, 0)))
        for dy in range(3):
            for dx in range(3):
                taps.append(xp[:, dy:dy + H, dx:dx + W, :])
    else:
        nc = len(pad_scratch)
        csz = C // nc
        for k in range(nc):
            pad_scratch[k][...] = jnp.pad(
                x[..., k * csz:(k + 1) * csz].astype(F32),
                ((0, 0), (1, 1), (1, 1), (0, 0)))
        for dy in range(3):
            for dx in range(3):
                parts = [
                    pad_scratch[k][:, dy:dy + 2 * (Ho - 1) + 1:2,
                                   dx:dx + 2 * (Wo - 1) + 1:2, :].astype(BF16)
                    for k in range(nc)]
                taps.append(parts[0] if nc == 1
                            else jnp.concatenate(parts, axis=-1))
    p = jnp.concatenate(taps, axis=-1).reshape(B * Ho * Wo, 9 * C)
    acc = jnp.dot(p, w, preferred_element_type=F32)
    acc = acc * s + b
    if res is not None:
        acc = acc + res
    if relu:
        acc = jnp.maximum(acc, 0.0)
    return acc.astype(BF16).reshape(B, Ho, Wo, w.shape[1])


def _basic_block(x, w1, s1, b1, w2, s2, b2, wd=None, sd=None, bd=None,
                 stride=1, pad_scratch=None):
    B, H, W, C = x.shape
    h = _conv3x3_bn(x, w1, s1, b1, stride=stride, pad_scratch=pad_scratch)
    _, Ho, Wo, Co = h.shape
    if wd is None:
        res = x.reshape(B * H * W, C).astype(F32)
    else:
        parts = [ps[:, 1:2 * (Ho - 1) + 2:2, 1:2 * (Wo - 1) + 2:2,
                    :].astype(BF16) for ps in pad_scratch]
        xs = parts[0] if len(parts) == 1 else jnp.concatenate(parts, axis=-1)
        res = jnp.dot(xs.reshape(B * Ho * Wo, C), wd,
                      preferred_element_type=F32) * sd + bd
    return _conv3x3_bn(h, w2, s2, b2, stride=1, res=res)


def _conv0_pool(p_ref, w_ref, s_ref, b_ref, yp_ref):
    B = yp_ref.shape[0]
    p = p_ref[...].reshape(B * 8192, 27)
    acc = jnp.dot(p, w_ref[...], preferred_element_type=F32)
    acc = jnp.maximum(acc * s_ref[...] + b_ref[...], 0.0)
    y = acc.reshape(B, 128, 64, 64)
    yp_ref[...] = jnp.pad(y, ((0, 0), (1, 1), (1, 1), (0, 0)))
    out = None
    for dy in range(3):
        for dx in range(3):
            t = yp_ref[:, dy:dy + 127:2, dx:dx + 63:2, :]
            out = t if out is None else jnp.maximum(out, t)
    return out.astype(BF16)


def _k2_body(p_ref, c_w, c_s, c_b,
             a_w1, a_s1, a_b1, a_w2, a_s2, a_b2,
             c_w1, c_s1, c_b1, c_w2, c_s2, c_b2,
             d_w1, d_s1, d_b1, d_w2, d_s2, d_b2, d_wd, d_sd, d_bd,
             e_w1, e_s1, e_b1, e_w2, e_s2, e_b2,
             o_ref, yp_ref, pad2_ref):
    B = o_ref.shape[0]
    y1 = _conv0_pool(p_ref, c_w, c_s, c_b, yp_ref)
    hb = B // 2
    for h in range(2):
        x = y1[h * hb:(h + 1) * hb]
        x = _basic_block(x, a_w1[...], a_s1[...], a_b1[...],
                         a_w2[...], a_s2[...], a_b2[...])
        x = _basic_block(x, c_w1[...], c_s1[...], c_b1[...],
                         c_w2[...], c_s2[...], c_b2[...])
        x = _basic_block(x, d_w1[...], d_s1[...], d_b1[...],
                         d_w2[...], d_s2[...], d_b2[...],
                         wd=d_wd[...], sd=d_sd[...], bd=d_bd[...],
                         stride=2,
                         pad_scratch=[pad2_ref.at[h * hb:(h + 1) * hb]])
        x = _basic_block(x, e_w1[...], e_s1[...], e_b1[...],
                         e_w2[...], e_s2[...], e_b2[...])
        o_ref[h * hb:(h + 1) * hb] = x


def _k3_body(x_ref,
             a_w1, a_s1, a_b1, a_w2, a_s2, a_b2, a_wd, a_sd, a_bd,
             c_w1, c_s1, c_b1, c_w2, c_s2, c_b2,
             d_w1, d_s1, d_b1, d_w2, d_s2, d_b2, d_wd, d_sd, d_bd,
             e_w1, e_s1, e_b1, e_w2, e_s2, e_b2,
             o_ref, pad3_ref, pad4a_ref, pad4b_ref):
    B = o_ref.shape[0]
    hb = B // 2
    for h in range(2):
        sl = slice(h * hb, (h + 1) * hb)
        x = x_ref[sl]
        x = _basic_block(x, a_w1[...], a_s1[...], a_b1[...],
                         a_w2[...], a_s2[...], a_b2[...],
                         wd=a_wd[...], sd=a_sd[...], bd=a_bd[...],
                         stride=2, pad_scratch=[pad3_ref.at[sl]])
        x = _basic_block(x, c_w1[...], c_s1[...], c_b1[...],
                         c_w2[...], c_s2[...], c_b2[...])
        x = _basic_block(x, d_w1[...], d_s1[...], d_b1[...],
                         d_w2[...], d_s2[...], d_b2[...],
                         wd=d_wd[...], sd=d_sd[...], bd=d_bd[...],
                         stride=2,
                         pad_scratch=[pad4a_ref.at[sl], pad4b_ref.at[sl]])
        x = _basic_block(x, e_w1[...], e_s1[...], e_b1[...],
                         e_w2[...], e_s2[...], e_b2[...])
        o_ref[sl] = jnp.mean(x.astype(F32), axis=(1, 2))


def _k4_body(x_ref, w1_ref, s1_ref, b1_ref, w2_ref, b2_ref, o_ref):
    h = jnp.dot(x_ref[...].astype(BF16), w1_ref[...],
                preferred_element_type=F32)
    h = jnp.maximum(h * s1_ref[...] + b1_ref[...], 0.0)
    out = jnp.dot(h.astype(BF16), w2_ref[...], preferred_element_type=F32)
    o_ref[...] = out + b2_ref[...]


def _wspec(shape):
    n = len(shape)
    return pl.BlockSpec(shape, lambda i, _n=n: (0,) * _n)


def kernel(x, conv0_w, conv0_s, conv0_b,
           layer1_0_w1, layer1_0_s1, layer1_0_b1,
           layer1_0_w2, layer1_0_s2, layer1_0_b2,
           layer1_1_w1, layer1_1_s1, layer1_1_b1,
           layer1_1_w2, layer1_1_s2, layer1_1_b2,
           layer2_0_w1, layer2_0_s1, layer2_0_b1,
           layer2_0_w2, layer2_0_s2, layer2_0_b2,
           layer2_0_wd, layer2_0_sd, layer2_0_bd,
           layer2_1_w1, layer2_1_s1, layer2_1_b1,
           layer2_1_w2, layer2_1_s2, layer2_1_b2,
           layer3_0_w1, layer3_0_s1, layer3_0_b1,
           layer3_0_w2, layer3_0_s2, layer3_0_b2,
           layer3_0_wd, layer3_0_sd, layer3_0_bd,
           layer3_1_w1, layer3_1_s1, layer3_1_b1,
           layer3_1_w2, layer3_1_s2, layer3_1_b2,
           layer4_0_w1, layer4_0_s1, layer4_0_b1,
           layer4_0_w2, layer4_0_s2, layer4_0_b2,
           layer4_0_wd, layer4_0_sd, layer4_0_bd,
           layer4_1_w1, layer4_1_s1, layer4_1_b1,
           layer4_1_w2, layer4_1_s2, layer4_1_b2,
           fc1_w, fc1_s, fc1_b, fc2_w, fc2_b):
    N = x.shape[0]

    xt = jnp.transpose(x, (0, 2, 3, 1)).astype(BF16)
    xpad = jnp.pad(xt, ((0, 0), (1, 1), (1, 1), (0, 0)))
    cols = [xpad[:, dy:dy + 128, dx:dx + 64, :]
            for dy in range(3) for dx in range(3)]
    p0 = jnp.concatenate(cols, axis=-1).reshape(N, 8192, 27)

    b2g = 2
    l2_args = [
        layer1_0_w1, layer1_0_s1, layer1_0_b1,
        layer1_0_w2, layer1_0_s2, layer1_0_b2,
        layer1_1_w1, layer1_1_s1, layer1_1_b1,
        layer1_1_w2, layer1_1_s2, layer1_1_b2,
        layer2_0_w1, layer2_0_s1, layer2_0_b1,
        layer2_0_w2, layer2_0_s2, layer2_0_b2,
        layer2_0_wd, layer2_0_sd, layer2_0_bd,
        layer2_1_w1, layer2_1_s1, layer2_1_b1,
        layer2_1_w2, layer2_1_s2, layer2_1_b2,
    ]
    y2 = pl.pallas_call(
        _k2_body,
        grid=(N // b2g,),
        in_specs=[pl.BlockSpec((b2g, 8192, 27), lambda i: (i, 0, 0)),
                  _wspec((27, 64)), _wspec((1, 64)), _wspec((1, 64))]
        + [_wspec(a.shape) for a in l2_args],
        out_specs=pl.BlockSpec((b2g, 32, 16, 128), lambda i: (i, 0, 0, 0)),
        out_shape=jax.ShapeDtypeStruct((N, 32, 16, 128), BF16),
        scratch_shapes=[pltpu.VMEM((b2g, 130, 66, 64), F32),
                        pltpu.VMEM((b2g, 66, 34, 64), F32)],
        compiler_params=_cparams(),
    )(p0, conv0_w, conv0_s, conv0_b, *l2_args)

    b3g = 8
    l3_args = [
        layer3_0_w1, layer3_0_s1, layer3_0_b1,
        layer3_0_w2, layer3_0_s2, layer3_0_b2,
        layer3_0_wd, layer3_0_sd, layer3_0_bd,
        layer3_1_w1, layer3_1_s1, layer3_1_b1,
        layer3_1_w2, layer3_1_s2, layer3_1_b2,
        layer4_0_w1, layer4_0_s1, layer4_0_b1,
        layer4_0_w2, layer4_0_s2, layer4_0_b2,
        layer4_0_wd, layer4_0_sd, layer4_0_bd,
        layer4_1_w1, layer4_1_s1, layer4_1_b1,
        layer4_1_w2, layer4_1_s2, layer4_1_b2,
    ]
    feat = pl.pallas_call(
        _k3_body,
        grid=(N // b3g,),
        in_specs=[pl.BlockSpec((b3g, 32, 16, 128), lambda i: (i, 0, 0, 0))]
        + [_wspec(a.shape) for a in l3_args],
        out_specs=pl.BlockSpec((b3g, 512), lambda i: (i, 0)),
        out_shape=jax.ShapeDtypeStruct((N, 512), F32),
        scratch_shapes=[pltpu.VMEM((b3g, 34, 18, 128), F32),
                        pltpu.VMEM((b3g, 18, 10, 128), F32),
                        pltpu.VMEM((b3g, 18, 10, 128), F32)],
        compiler_params=_cparams(),
    )(y2, *l3_args)

    tm = N // 2
    logits = pl.pallas_call(
        _k4_body,
        grid=(2,),
        in_specs=[
            pl.BlockSpec((tm, 512), lambda i: (i, 0)),
            _wspec(fc1_w.shape), _wspec(fc1_s.shape), _wspec(fc1_b.shape),
            _wspec(fc2_w.shape), _wspec(fc2_b.shape),
        ],
        out_specs=pl.BlockSpec((tm, fc2_w.shape[1]), lambda i: (i, 0)),
        out_shape=jax.ShapeDtypeStruct((N, fc2_w.shape[1]), F32),
        compiler_params=_cparams(),
    )(feat, fc1_w, fc1_s, fc1_b, fc2_w, fc2_b)

    return logits[:, :NUM_CLASSES]
